# baseline (speedup 1.0000x reference)
import jax
import jax.numpy as jnp
from jax.experimental import pallas as pl
from jax.experimental.pallas import tpu as pltpu

_T = 10000
_K = 20


def _phase1(zx_ref, ze_ref, a1x_ref, a1e_ref, b1big_ref, a2_ref, b2vec_ref,
            xe_ref, part_ref):
    nb = zx_ref.shape[0]
    hs = [jax.lax.dot_general(zx_ref[b, 0], a1x_ref[...],
                              (((1,), (0,)), ((), ())),
                              preferred_element_type=jnp.float32)
          for b in range(nb)]
    hs += [jax.lax.dot_general(ze_ref[b, 0], a1e_ref[...],
                               (((1,), (0,)), ((), ())),
                               preferred_element_type=jnp.float32)
           for b in range(nb)]
    hbig = jnp.tanh(jnp.concatenate(hs, axis=1) + b1big_ref[...])
    xe = jnp.tanh(
        jax.lax.dot_general(hbig, a2_ref[...], (((1,), (0,)), ((), ())),
                            preferred_element_type=jnp.float32)
        + b2vec_ref[...])
    xe_ref[0] = xe

    colmin = jnp.min(xe, axis=0, keepdims=True)
    colmax = jnp.max(xe, axis=0, keepdims=True)
    colsum = jnp.sum(xe, axis=0, keepdims=True)
    colsq = jnp.sum(xe * xe, axis=0, keepdims=True)
    lane = jax.lax.broadcasted_iota(jnp.int32, (1, 128), 1)
    isx = lane < 64
    pinf = jnp.float32(jnp.inf)
    for b in range(nb):
        m = isx & (lane % 4 == b)
        part_ref[0, 0, b] = jnp.min(jnp.where(m, colmin, pinf))
        part_ref[0, 0, nb + b] = jnp.max(jnp.where(m, colmax, -pinf))
    zero = jnp.float32(0.0)
    part_ref[0, 0, 2 * nb] = jnp.sum(jnp.where(isx, zero, colsum))
    part_ref[0, 0, 2 * nb + 1] = jnp.sum(jnp.where(isx, zero, colsq))


def _phase2(xe_ref, amat_ref, wmat_ref, smat_ref, base_ref, alpha_ref, y_ref):
    xe = xe_ref[0]
    acc = base_ref[...] + jnp.zeros_like(xe)
    for s in range(_K - 1):
        t = xe - amat_ref[s:s + 1, :]
        t = jnp.maximum(jnp.minimum(t, wmat_ref[s:s + 1, :]),
                        jnp.float32(0.0))
        acc = acc + smat_ref[s:s + 1, :] * t
    y_ref[0] = acc[:, 0:64] + alpha_ref[0, 0] * xe[:, 64:128]


def kernel(zf, zx, ze, Wf1, bf1, Wf2, bf2, Wx1, bx1, Wx2, bx2,
           We1, be1, We2, be2):
    B, N, L = zx.shape
    T = _T
    C = N // T
    R = T // 16
    f32 = jnp.float32

    pts = jnp.tanh(jnp.tanh(zf @ Wf1 + bf1) @ Wf2 + bf2)
    K = pts.shape[1]
    dirs = jax.random.randint(jax.random.key(42), (B,), 0, 2).astype(bool)
    srt = jnp.sort(pts, axis=1)
    yp = jnp.where(dirs[:, None], srt, srt[:, ::-1])
    xp = jnp.linspace(0.0, 1.0, K).astype(f32)

    zx5 = zx.reshape(B, C, R, 16 * L)
    ze5 = ze.reshape(B, C, R, 16 * L)
    i16 = jnp.eye(16, dtype=f32)
    a1x = jnp.kron(i16, Wx1)
    a1e = jnp.kron(i16, We1)
    b1big = jnp.concatenate(
        [jnp.tile(bx1, 16)] * B + [jnp.tile(be1, 16)] * B).reshape(1, B * 256)
    i4 = jnp.eye(B, dtype=f32)
    colsx = jnp.concatenate(
        [jnp.kron(i16, jnp.outer(Wx2[:, 0], i4[b])) for b in range(B)],
        axis=0)
    colse = jnp.concatenate(
        [jnp.kron(i16, jnp.outer(We2[:, 0], i4[b])) for b in range(B)],
        axis=0)
    z64 = jnp.zeros((B * 128, 64), f32)
    a2 = jnp.concatenate(
        [jnp.concatenate([colsx, z64], axis=1),
         jnp.concatenate([z64, colse], axis=1)], axis=0)
    b2vec = jnp.concatenate(
        [jnp.full((64,), bx2[0], f32), jnp.full((64,), be2[0], f32)]
    ).reshape(1, 128)

    full = lambda shp: pl.BlockSpec(shp, lambda i: (0,) * len(shp))
    chunk_spec = pl.BlockSpec((B, 1, R, 128), lambda i: (0, i, 0, 0))
    xe_spec = pl.BlockSpec((1, R, 128), lambda i: (i, 0, 0))

    xebuf, parts = pl.pallas_call(
        _phase1,
        grid=(C,),
        in_specs=[
            chunk_spec, chunk_spec,
            full((128, 128)), full((128, 128)), full((1, B * 256)),
            full((B * 256, 128)), full((1, 128)),
        ],
        out_specs=[
            xe_spec,
            pl.BlockSpec((1, 1, 16), lambda i: (i, 0, 0),
                         memory_space=pltpu.SMEM),
        ],
        out_shape=[
            jax.ShapeDtypeStruct((C, R, 128), f32),
            jax.ShapeDtypeStruct((C, 1, 16), f32),
        ],
    )(zx5, ze5, a1x, a1e, b1big, a2, b2vec)

    parts = parts[:, 0, :]
    mn = jnp.min(parts[:, 0:B], axis=0)
    mx = jnp.max(parts[:, B:2 * B], axis=0)
    S = jnp.sum(parts[:, 2 * B])
    S2 = jnp.sum(parts[:, 2 * B + 1])
    ntot = f32(B * N)
    mean = S / ntot
    var = (S2 - S * S / ntot) / (ntot - f32(1.0))
    std = jnp.sqrt(var)
    alpha = (f32(0.1) / std).reshape(1, 1)
    beta = -f32(0.1) * mean / std

    D = (mx - mn)[:, None]
    w = (xp[1:] - xp[:-1])[None, :]
    slope = (yp[:, 1:] - yp[:, :-1]) / (w + f32(1e-7))
    a_tbl = mn[:, None] + xp[None, :K - 1] * D
    w_tbl = w * D
    s_tbl = slope / D
    base = yp[:, 0] + beta

    idx = jnp.arange(128)
    bb = idx % B
    isx = (idx < 64)[None, :]
    zmat = jnp.zeros((K - 1, 128), f32)
    amat = jnp.where(isx, a_tbl.T[:, bb], zmat)
    wmat = jnp.where(isx, w_tbl.T[:, bb], zmat)
    smat = jnp.where(isx, s_tbl.T[:, bb], zmat)
    basev = jnp.where(isx[0], base[bb], f32(0.0)).reshape(1, 128)

    ybuf = pl.pallas_call(
        _phase2,
        grid=(C,),
        in_specs=[
            xe_spec,
            full((K - 1, 128)), full((K - 1, 128)), full((K - 1, 128)),
            full((1, 128)),
            pl.BlockSpec((1, 1), lambda i: (0, 0), memory_space=pltpu.SMEM),
        ],
        out_specs=pl.BlockSpec((1, R, 64), lambda i: (i, 0, 0)),
        out_shape=jax.ShapeDtypeStruct((C, R, 64), f32),
    )(xebuf, amat, wmat, smat, basev, alpha)

    return ybuf.reshape(N, B)


# speedup vs baseline: 2.6797x; 2.6797x over previous
import jax
import jax.numpy as jnp
from jax.experimental import pallas as pl
from jax.experimental.pallas import tpu as pltpu

_T = 2000
_K = 20


def _stream(z_ref, w1_ref, b1t_ref, m2_ref, b2_ref):
    nb = z_ref.shape[0]
    h = jnp.concatenate(
        [jax.lax.dot_general(w1_ref[...], z_ref[b, 0],
                             (((0,), (1,)), ((), ())),
                             preferred_element_type=jnp.float32)
         for b in range(nb)], axis=0)
    h = jnp.tanh(h + b1t_ref[...])
    return jnp.tanh(
        jax.lax.dot_general(m2_ref[...], h, (((1,), (0,)), ((), ())),
                            preferred_element_type=jnp.float32)
        + b2_ref[...])


def _fused(zx_ref, ze_ref, wx1_ref, bx1t_ref, m2x_ref, bx2_ref,
           we1_ref, be1t_ref, m2e_ref, be2_ref, yp_ref,
           y_ref, xs_ref, es_ref, acc_ref):
    p = pl.program_id(0)
    i = pl.program_id(1)
    nb = 4
    f32 = jnp.float32

    @pl.when(p == 0)
    def _phase0():
        x4 = _stream(zx_ref, wx1_ref, bx1t_ref, m2x_ref, bx2_ref)
        e4 = _stream(ze_ref, we1_ref, be1t_ref, m2e_ref, be2_ref)
        xs_ref[i] = x4
        es_ref[i] = e4
        first = i == 0
        for b in range(nb):
            mnb = jnp.min(x4[b:b + 1, :])
            mxb = jnp.max(x4[b:b + 1, :])
            acc_ref[0, b] = jnp.where(first, mnb,
                                      jnp.minimum(acc_ref[0, b], mnb))
            acc_ref[0, nb + b] = jnp.where(first, mxb,
                                           jnp.maximum(acc_ref[0, nb + b],
                                                       mxb))
        se = jnp.sum(e4)
        sq = jnp.sum(e4 * e4)
        acc_ref[0, 2 * nb] = jnp.where(first, se, acc_ref[0, 2 * nb] + se)
        acc_ref[0, 2 * nb + 1] = jnp.where(first, sq,
                                           acc_ref[0, 2 * nb + 1] + sq)

    @pl.when(p == 1)
    def _phase1():
        T = xs_ref.shape[2]
        C = xs_ref.shape[0]
        ntot = f32(nb * C * T)
        S = acc_ref[0, 2 * nb]
        S2 = acc_ref[0, 2 * nb + 1]
        mean = S / ntot
        std = jnp.sqrt((S2 - S * S / ntot) / (ntot - f32(1.0)))
        alpha = f32(0.1) / std
        beta = -f32(0.1) * mean / std

        mn_col = jnp.concatenate(
            [jnp.full((1, 1), acc_ref[0, b], f32) for b in range(nb)], axis=0)
        mx_col = jnp.concatenate(
            [jnp.full((1, 1), acc_ref[0, nb + b], f32) for b in range(nb)],
            axis=0)
        d_col = mx_col - mn_col
        inv_d = f32(1.0) / d_col

        x4 = xs_ref[i]
        e4 = es_ref[i]
        h = 1.0 / (_K - 1)
        xpv = [s * h for s in range(_K)]
        y = yp_ref[:, 0:1] + (beta + alpha * e4)
        for s in range(_K - 1):
            ws = f32(xpv[s + 1] - xpv[s])
            a_col = mn_col + f32(xpv[s]) * d_col
            w_col = ws * d_col
            s_col = ((yp_ref[:, s + 1:s + 2] - yp_ref[:, s:s + 1])
                     / (ws + f32(1e-7))) * inv_d
            t = x4 - a_col
            t = jnp.maximum(jnp.minimum(t, w_col), f32(0.0))
            y = y + s_col * t
        y_ref[0] = y


def kernel(zf, zx, ze, Wf1, bf1, Wf2, bf2, Wx1, bx1, Wx2, bx2,
           We1, be1, We2, be2):
    B, N, L = zx.shape
    T = _T
    C = N // T
    f32 = jnp.float32

    pts = jnp.tanh(jnp.tanh(zf @ Wf1 + bf1) @ Wf2 + bf2)
    K = pts.shape[1]
    dirs = jax.random.randint(jax.random.key(42), (B,), 0, 2).astype(bool)
    srt = jnp.sort(pts, axis=1)
    yp = jnp.where(dirs[:, None], srt, srt[:, ::-1])

    zx4 = zx.reshape(B, C, T, L)
    ze4 = ze.reshape(B, C, T, L)
    bx1t = jnp.tile(bx1, B).reshape(B * L, 1).astype(f32)
    be1t = jnp.tile(be1, B).reshape(B * L, 1).astype(f32)
    eye = jnp.eye(B, dtype=f32)
    m2x = jnp.kron(eye, Wx2[:, 0][None, :])
    m2e = jnp.kron(eye, We2[:, 0][None, :])
    bx2c = bx2.reshape(1, 1)
    be2c = be2.reshape(1, 1)

    full = lambda shp: pl.BlockSpec(shp, lambda p, i: (0,) * len(shp))
    chunk_spec = pl.BlockSpec((B, 1, T, L),
                              lambda p, i: (0, i * (1 - p), 0, 0))

    ybuf = pl.pallas_call(
        _fused,
        grid=(2, C),
        in_specs=[
            chunk_spec, chunk_spec,
            full((L, L)), full((B * L, 1)), full((B, B * L)), full((1, 1)),
            full((L, L)), full((B * L, 1)), full((B, B * L)), full((1, 1)),
            full((B, K)),
        ],
        out_specs=pl.BlockSpec((1, B, T), lambda p, i: (i * p, 0, 0)),
        out_shape=jax.ShapeDtypeStruct((C, B, T), f32),
        scratch_shapes=[
            pltpu.VMEM((C, B, T), f32),
            pltpu.VMEM((C, B, T), f32),
            pltpu.SMEM((1, 16), f32),
        ],
    )(zx4, ze4, Wx1, bx1t, m2x, bx2c, We1, be1t, m2e, be2c, yp)

    return ybuf.transpose(0, 2, 1).reshape(N, B)
